# baseline (speedup 1.0000x reference)
import functools

import jax
import jax.numpy as jnp
from jax import lax
from jax.experimental import pallas as pl
from jax.experimental.pallas import tpu as pltpu
from jax.experimental.pallas import tpu_sc as plsc

NC = 2
NS = 16
NW = NC * NS
L = 16


def _pack_half_rows(w):
    n, d = w.shape
    blk = 512
    grid = pl.cdiv(n, 2 * blk)

    def body(x_ref, o_ref):
        x = x_ref[...]
        o_ref[...] = jnp.concatenate([x[:blk], x[blk:]], axis=1)

    return pl.pallas_call(
        body,
        grid=(grid,),
        in_specs=[pl.BlockSpec((2 * blk, d), lambda i: (i, 0))],
        out_specs=pl.BlockSpec((blk, 2 * d), lambda i: (i, 0)),
        out_shape=jax.ShapeDtypeStruct((grid * blk, 2 * d), w.dtype),
    )(w)


def _transpose_half(o):
    d, n = o.shape
    blk = 512
    grid = pl.cdiv(n, 2 * blk)

    def body(x_ref, o_ref):
        t = x_ref[...].T
        o_ref[...] = jnp.concatenate([t[:blk], t[blk:]], axis=1)

    return pl.pallas_call(
        body,
        grid=(grid,),
        in_specs=[pl.BlockSpec((d, 2 * blk), lambda i: (0, i))],
        out_specs=pl.BlockSpec((blk, 2 * d), lambda i: (i, 0)),
        out_shape=jax.ShapeDtypeStruct((grid * blk, 2 * d), o.dtype),
    )(o)


def _split_row(w):
    return (lax.shift_left(lax.shift_right_logical(w, 10), 9)
            + lax.bitwise_and(w, 511))


def _split_col(w):
    return lax.bitwise_and(lax.shift_right_logical(w, 3), 64)


def _sc_forward(ids, pmat, w2, ot2, C, S):
    BW = 128
    D = pmat.shape[1]
    KI = ids.shape[1]
    CCH = 2
    NCC = C // CCH
    PG = BW // L
    mesh = plsc.VectorSubcoreMesh(core_axis_name="c", subcore_axis_name="s")

    @functools.partial(
        pl.kernel,
        out_type=jax.ShapeDtypeStruct((NW, 8, 128), jnp.float32),
        mesh=mesh,
        scratch_types=[
            pltpu.VMEM((KI, 128), jnp.int32),
            pltpu.VMEM((C, 128), jnp.int32),
            pltpu.VMEM((S, 128), jnp.int32),
            pltpu.VMEM((L, 8, D), jnp.float32),
            pltpu.VMEM((L, 8, D), jnp.float32),
            pltpu.VMEM((BW * D,), jnp.float32),
            pltpu.VMEM((CCH * 128, 128), jnp.float32),
            pltpu.VMEM((CCH * 128, 128), jnp.float32),
            pltpu.VMEM((D, 128), jnp.float32),
            pltpu.VMEM((8, 128), jnp.float32),
            pltpu.SemaphoreType.DMA,
            pltpu.SemaphoreType.DMA,
            pltpu.SemaphoreType.DMA,
            pltpu.SemaphoreType.DMA,
        ],
        compiler_params=pltpu.CompilerParams(
            needs_layout_passes=False, use_tc_tiling_on_sc=True
        ),
    )
    def k(ids_hbm, p_hbm, w2_hbm, ot2_hbm, out_hbm,
          idsv, cidx, sidx, ptA, ptB, prow, rowsA, rowsB, acc, res,
          semPA, semPB, semA, semB):
        wid = lax.axis_index("c") * NS + lax.axis_index("s")
        iota = lax.iota(jnp.int32, L)

        pltpu.sync_copy(ids_hbm.at[wid], idsv)

        @pl.loop(0, C)
        def _(c):
            @pl.loop(0, 128 // L)
            def _(j):
                w = idsv[1 + c, pl.ds(j * L, L)]
                cidx[c, pl.ds(j * L, L)] = _split_row(w)

        @pl.loop(0, S)
        def _(sj):
            @pl.loop(0, 128 // L)
            def _(j):
                w = idsv[1 + C + sj, pl.ds(j * L, L)]
                sidx[sj, pl.ds(j * L, L)] = _split_row(w)

        ctx_sems = [semA, semB]
        ctx_bufs = [rowsA, rowsB]

        def fire_ctx(cc):
            buf = ctx_bufs[cc % 2]
            sem = ctx_sems[cc % 2]
            return [
                pltpu.async_copy(
                    w2_hbm.at[cidx.at[cc * CCH + cl]],
                    buf.at[pl.ds(cl * 128, 128)],
                    sem,
                )
                for cl in range(CCH)
            ]

        pending_ctx = fire_ctx(0)

        pt_bufs = [ptA, ptB]
        pt_sems = [semPA, semPB]

        def fire_pt(g):
            docv = idsv[0, pl.ds(g * L, L)]
            buf = pt_bufs[g % 2]
            sem = pt_sems[g % 2]
            cps = []
            for t in range(L):
                start = pl.multiple_of(lax.bitwise_and(docv[t], -8), 8)
                cps.append(
                    pltpu.async_copy(p_hbm.at[pl.ds(start, 8)], buf.at[t], sem)
                )
            return cps

        pending_pt = fire_pt(0)
        for g in range(PG):
            nxt = fire_pt(g + 1) if g + 1 < PG else None
            for cp in pending_pt:
                cp.wait()
            pending_pt = nxt
            buf = pt_bufs[g % 2]
            b0 = g * L
            rvec = lax.bitwise_and(idsv[0, pl.ds(b0, L)], 7)
            b64 = (iota + b0) * D

            @pl.loop(0, D)
            def _(d, buf=buf, rvec=rvec, b64=b64):
                dv = jnp.full((L,), d, dtype=jnp.int32)
                v = plsc.load_gather(buf, [iota, rvec, dv])
                plsc.store_scatter(prow, [b64 + dv], v)

        for cc in range(NCC):
            nxt = fire_ctx(cc + 1) if cc + 1 < NCC else None
            for cp in pending_ctx:
                cp.wait()
            pending_ctx = nxt
            buf = ctx_bufs[cc % 2]

            @pl.loop(0, BW // L)
            def _(bg, cc=cc, buf=buf):
                b0 = bg * L
                b64 = (iota + b0) * D
                rowvecs = []
                colbases = []
                for cl in range(CCH):
                    w = idsv[1 + cc * CCH + cl, pl.ds(b0, L)]
                    rowvecs.append(iota + (cl * 128 + b0))
                    colbases.append(_split_col(w))

                @pl.loop(0, D)
                def _(d, cc=cc, b0=b0, b64=b64,
                      rowvecs=rowvecs, colbases=colbases):
                    dv = jnp.full((L,), d, dtype=jnp.int32)
                    if cc == 0:
                        v = plsc.load_gather(prow, [b64 + dv])
                    else:
                        v = acc[d, pl.ds(b0, L)]
                    for cl in range(CCH):
                        v = v + plsc.load_gather(
                            buf, [rowvecs[cl], colbases[cl] + dv]
                        )
                    acc[d, pl.ds(b0, L)] = v

        NSC = pl.cdiv(S, CCH)

        def fire_smp(sc):
            nr = min(CCH, S - sc * CCH)
            buf = ctx_bufs[sc % 2]
            sem = ctx_sems[sc % 2]
            return [
                pltpu.async_copy(
                    ot2_hbm.at[sidx.at[sc * CCH + j]],
                    buf.at[pl.ds(j * 128, 128)],
                    sem,
                )
                for j in range(nr)
            ]

        pending_smp = fire_smp(0)
        for sc in range(NSC):
            nxt = fire_smp(sc + 1) if sc + 1 < NSC else None
            for cp in pending_smp:
                cp.wait()
            pending_smp = nxt
            buf = ctx_bufs[sc % 2]
            nr = min(CCH, S - sc * CCH)

            @pl.loop(0, nr * (128 // L))
            def _(gl, sc=sc, buf=buf):
                g = sc * CCH * (128 // L) + gl
                r8 = g // (128 // L)
                c8 = g - r8 * (128 // L)
                w = idsv[1 + C + r8, pl.ds(c8 * L, L)]
                colb = _split_col(w)
                ivec = iota + gl * L
                bvec = (iota + g * L) // S

                def dbody(t, r):
                    for dd in range(4):
                        d = t * 4 + dd
                        dv = jnp.full((L,), d, dtype=jnp.int32)
                        a = plsc.load_gather(acc, [dv, bvec])
                        o = plsc.load_gather(buf, [ivec, colb + dv])
                        r = r + a * o
                    return r

                r = lax.fori_loop(0, D // 4, dbody,
                                  jnp.zeros((L,), jnp.float32))
                res[r8, pl.ds(c8 * L, L)] = r

        pltpu.sync_copy(res, out_hbm.at[wid])

    return k(ids, pmat, w2, ot2)


def kernel(doc_ids, context_ids, sample_ids, paragraph_matrix, word_matrix,
           outputs):
    B, = doc_ids.shape
    C = context_ids.shape[1]
    S = sample_ids.shape[1]
    BW = B // NW
    w2 = _pack_half_rows(word_matrix)
    ot2 = _transpose_half(outputs)
    doc_b = doc_ids.reshape(NW, 1, BW)
    ctx_b = context_ids.reshape(NW, BW, C).transpose(0, 2, 1)
    smp_b = sample_ids.reshape(NW, S, BW)
    pad = jnp.zeros((NW, 32 - 1 - C - S, BW), jnp.int32)
    ids = jnp.concatenate([doc_b, ctx_b, smp_b, pad], axis=1)
    res = _sc_forward(ids, paragraph_matrix, w2, ot2, C, S)
    return res[:, : S * BW // 128, :].reshape(B, S)


# speedup vs baseline: 2.5309x; 2.5309x over previous
import functools

import jax
import jax.numpy as jnp
from jax import lax
from jax.experimental import pallas as pl
from jax.experimental.pallas import tpu as pltpu
from jax.experimental.pallas import tpu_sc as plsc

NC = 2
NS = 16
NW = NC * NS
L = 16
KB = 11


def _transpose_pack(o):
    d, n = o.shape
    hb = 1 << KB
    grid = pl.cdiv(n, 2 * hb)

    def body(x_ref, o_ref):
        t = x_ref[...].T
        o_ref[...] = jnp.concatenate([t[:hb], t[hb:]], axis=1)

    return pl.pallas_call(
        body,
        grid=(grid,),
        in_specs=[pl.BlockSpec((d, 2 * hb), lambda i: (0, i))],
        out_specs=pl.BlockSpec((hb, 2 * d), lambda i: (i, 0)),
        out_shape=jax.ShapeDtypeStruct((grid * hb, 2 * d), o.dtype),
    )(o)


def _split_row(w):
    return (lax.shift_left(lax.shift_right_logical(w, KB + 1), KB)
            + lax.bitwise_and(w, (1 << KB) - 1))


def _split_col(w):
    return lax.shift_left(
        lax.bitwise_and(lax.shift_right_logical(w, KB), 1), 6)


_SC_PARAMS = pltpu.CompilerParams(
    needs_layout_passes=False, use_tc_tiling_on_sc=True
)
_MESH = plsc.VectorSubcoreMesh(core_axis_name="c", subcore_axis_name="s")


def _sc_fetch_p(ids, pt):
    D = pt.shape[0]
    BW = 128

    @functools.partial(
        pl.kernel,
        out_type=jax.ShapeDtypeStruct((NW, 8, 1024), jnp.float32),
        mesh=_MESH,
        scratch_types=[
            pltpu.VMEM((BW,), jnp.int32),
            pltpu.VMEM((D, 128), jnp.float32),
            pltpu.VMEM((D, 128), jnp.float32),
            pltpu.VMEM((8, 1024), jnp.float32),
            pltpu.SemaphoreType.DMA,
            pltpu.SemaphoreType.DMA,
        ],
        compiler_params=_SC_PARAMS,
    )
    def k(ids_hbm, pt_hbm, out_hbm, idv, bufA, bufB, prow, semA, semB):
        wid = lax.axis_index("c") * NS + lax.axis_index("s")
        iota = lax.iota(jnp.int32, L)
        pltpu.sync_copy(ids_hbm.at[wid, 0], idv)

        bufs = [bufA, bufB]
        sems = [semA, semB]
        docvs = [idv[pl.ds(g * L, L)] for g in range(BW // L)]

        def fire(b):
            doc = docvs[b // L][b % L]
            start = pl.multiple_of(lax.bitwise_and(doc, -128), 128)
            return pltpu.async_copy(
                pt_hbm.at[:, pl.ds(start, 128)], bufs[b % 2], sems[b % 2]
            )

        pend = fire(0)
        for b in range(BW):
            nxt = fire(b + 1) if b + 1 < BW else None
            pend.wait()
            pend = nxt
            buf = bufs[b % 2]
            q = lax.bitwise_and(docvs[b // L][b % L], 127)
            qv = jnp.full((L,), q, dtype=jnp.int32)
            for kk in range(D // L):
                v = plsc.load_gather(buf, [iota + kk * L, qv])
                flat = b * D + kk * L
                prow[flat // 1024, pl.ds(flat % 1024, L)] = v

        pltpu.sync_copy(prow, out_hbm.at[wid])

    return k(ids, pt)


def _sc_main(ids, pr, w2, ot2, C, S):
    BW = 128
    D = 64
    KI = ids.shape[1]
    CCH = 2
    NCC = C // CCH

    @functools.partial(
        pl.kernel,
        out_type=jax.ShapeDtypeStruct((NW, 8, 128), jnp.float32),
        mesh=_MESH,
        scratch_types=[
            pltpu.VMEM((KI, 128), jnp.int32),
            pltpu.VMEM((C, 128), jnp.int32),
            pltpu.VMEM((S, 128), jnp.int32),
            pltpu.VMEM((BW * D,), jnp.float32),
            pltpu.VMEM((CCH * 128, 128), jnp.float32),
            pltpu.VMEM((CCH * 128, 128), jnp.float32),
            pltpu.VMEM((D, 128), jnp.float32),
            pltpu.VMEM((8, 128), jnp.float32),
            pltpu.SemaphoreType.DMA,
            pltpu.SemaphoreType.DMA,
        ],
        compiler_params=_SC_PARAMS,
    )
    def k(ids_hbm, pr_hbm, w2_hbm, ot2_hbm, out_hbm,
          idsv, cidx, sidx, prow, rowsA, rowsB, acc, res, semA, semB):
        wid = lax.axis_index("c") * NS + lax.axis_index("s")
        iota = lax.iota(jnp.int32, L)

        pltpu.sync_copy(ids_hbm.at[wid], idsv)

        @pl.loop(0, C)
        def _(c):
            @pl.loop(0, 128 // L)
            def _(j):
                w = idsv[1 + c, pl.ds(j * L, L)]
                cidx[c, pl.ds(j * L, L)] = _split_row(w)

        @pl.loop(0, S)
        def _(sj):
            @pl.loop(0, 128 // L)
            def _(j):
                w = idsv[1 + C + sj, pl.ds(j * L, L)]
                sidx[sj, pl.ds(j * L, L)] = _split_row(w)

        ctx_sems = [semA, semB]
        ctx_bufs = [rowsA, rowsB]

        def fire_ctx(cc):
            buf = ctx_bufs[cc % 2]
            sem = ctx_sems[cc % 2]
            return [
                pltpu.async_copy(
                    w2_hbm.at[cidx.at[cc * CCH + cl]],
                    buf.at[pl.ds(cl * 128, 128)],
                    sem,
                )
                for cl in range(CCH)
            ]

        pending_ctx = fire_ctx(0)

        for r in range(8):
            pltpu.sync_copy(pr_hbm.at[wid, r], prow.at[pl.ds(r * 1024, 1024)])

        for cc in range(NCC):
            nxt = fire_ctx(cc + 1) if cc + 1 < NCC else None
            for cp in pending_ctx:
                cp.wait()
            pending_ctx = nxt
            buf = ctx_bufs[cc % 2]

            @pl.loop(0, BW // L)
            def _(bg, cc=cc, buf=buf):
                b0 = bg * L
                b64 = (iota + b0) * D
                rowvecs = []
                colbases = []
                for cl in range(CCH):
                    w = idsv[1 + cc * CCH + cl, pl.ds(b0, L)]
                    rowvecs.append(iota + (cl * 128 + b0))
                    colbases.append(_split_col(w))

                @pl.loop(0, D)
                def _(d, cc=cc, b0=b0, b64=b64,
                      rowvecs=rowvecs, colbases=colbases):
                    dv = jnp.full((L,), d, dtype=jnp.int32)
                    if cc == 0:
                        v = plsc.load_gather(prow, [b64 + dv])
                    else:
                        v = acc[d, pl.ds(b0, L)]
                    for cl in range(CCH):
                        v = v + plsc.load_gather(
                            buf, [rowvecs[cl], colbases[cl] + dv]
                        )
                    acc[d, pl.ds(b0, L)] = v

        NSC = pl.cdiv(S, CCH)

        def fire_smp(sc):
            nr = min(CCH, S - sc * CCH)
            buf = ctx_bufs[sc % 2]
            sem = ctx_sems[sc % 2]
            return [
                pltpu.async_copy(
                    ot2_hbm.at[sidx.at[sc * CCH + j]],
                    buf.at[pl.ds(j * 128, 128)],
                    sem,
                )
                for j in range(nr)
            ]

        pending_smp = fire_smp(0)
        for sc in range(NSC):
            nxt = fire_smp(sc + 1) if sc + 1 < NSC else None
            for cp in pending_smp:
                cp.wait()
            pending_smp = nxt
            buf = ctx_bufs[sc % 2]
            nr = min(CCH, S - sc * CCH)

            @pl.loop(0, nr * (128 // L))
            def _(gl, sc=sc, buf=buf):
                g = sc * CCH * (128 // L) + gl
                r8 = g // (128 // L)
                c8 = g - r8 * (128 // L)
                w = idsv[1 + C + r8, pl.ds(c8 * L, L)]
                colb = _split_col(w)
                ivec = iota + gl * L
                bvec = (iota + g * L) // S

                def dbody(t, r):
                    for dd in range(4):
                        d = t * 4 + dd
                        dv = jnp.full((L,), d, dtype=jnp.int32)
                        a = plsc.load_gather(acc, [dv, bvec])
                        o = plsc.load_gather(buf, [ivec, colb + dv])
                        r = r + a * o
                    return r

                r = lax.fori_loop(0, D // 4, dbody,
                                  jnp.zeros((L,), jnp.float32))
                res[r8, pl.ds(c8 * L, L)] = r

        pltpu.sync_copy(res, out_hbm.at[wid])

    return k(ids, pr, w2, ot2)


def kernel(doc_ids, context_ids, sample_ids, paragraph_matrix, word_matrix,
           outputs):
    B, = doc_ids.shape
    C = context_ids.shape[1]
    S = sample_ids.shape[1]
    BW = B // NW
    w2 = _transpose_pack(word_matrix.T)
    ot2 = _transpose_pack(outputs)
    doc_b = doc_ids.reshape(NW, 1, BW)
    ctx_b = context_ids.reshape(NW, BW, C).transpose(0, 2, 1)
    smp_b = sample_ids.reshape(NW, S, BW)
    pad = jnp.zeros((NW, 32 - 1 - C - S, BW), jnp.int32)
    ids = jnp.concatenate([doc_b, ctx_b, smp_b, pad], axis=1)
    pr = _sc_fetch_p(ids, paragraph_matrix.T)
    res = _sc_main(ids, pr, w2, ot2, C, S)
    return res[:, : S * BW // 128, :].reshape(B, S)


# speedup vs baseline: 3.4894x; 1.3787x over previous
import functools

import jax
import jax.numpy as jnp
from jax import lax
from jax.experimental import pallas as pl
from jax.experimental.pallas import tpu as pltpu
from jax.experimental.pallas import tpu_sc as plsc

NC = 2
NS = 16
NW = NC * NS
L = 16
KB = 12


def _transpose_pack(o):
    d, n = o.shape
    hb = 1 << KB
    grid = pl.cdiv(n, 2 * hb)

    def body(x_ref, o_ref):
        t = x_ref[...].T
        o_ref[...] = jnp.concatenate([t[:hb], t[hb:]], axis=1)

    return pl.pallas_call(
        body,
        grid=(grid,),
        in_specs=[pl.BlockSpec((d, 2 * hb), lambda i: (0, i))],
        out_specs=pl.BlockSpec((hb, 2 * d), lambda i: (i, 0)),
        out_shape=jax.ShapeDtypeStruct((grid * hb, 2 * d), o.dtype),
    )(o)


def _split_row(w):
    return (lax.shift_left(lax.shift_right_logical(w, KB + 1), KB)
            + lax.bitwise_and(w, (1 << KB) - 1))


def _split_col(w):
    return lax.shift_left(
        lax.bitwise_and(lax.shift_right_logical(w, KB), 1), 6)


_SC_PARAMS = pltpu.CompilerParams(
    needs_layout_passes=False, use_tc_tiling_on_sc=True
)
_MESH = plsc.VectorSubcoreMesh(core_axis_name="c", subcore_axis_name="s")


def _sc_fetch_p(ids, pt):
    D = pt.shape[0]
    BW = 128

    @functools.partial(
        pl.kernel,
        out_type=jax.ShapeDtypeStruct((NW, 8, 1024), jnp.float32),
        mesh=_MESH,
        scratch_types=[
            pltpu.VMEM((BW,), jnp.int32),
            pltpu.VMEM((D, 128), jnp.float32),
            pltpu.VMEM((D, 128), jnp.float32),
            pltpu.VMEM((8, 1024), jnp.float32),
            pltpu.SemaphoreType.DMA,
            pltpu.SemaphoreType.DMA,
        ],
        compiler_params=_SC_PARAMS,
    )
    def k(ids_hbm, pt_hbm, out_hbm, idv, bufA, bufB, prow, semA, semB):
        wid = lax.axis_index("c") * NS + lax.axis_index("s")
        iota = lax.iota(jnp.int32, L)
        pltpu.sync_copy(ids_hbm.at[wid, 0], idv)

        bufs = [bufA, bufB]
        sems = [semA, semB]
        docvs = [idv[pl.ds(g * L, L)] for g in range(BW // L)]

        def fire(b):
            doc = docvs[b // L][b % L]
            start = pl.multiple_of(lax.bitwise_and(doc, -128), 128)
            return pltpu.async_copy(
                pt_hbm.at[:, pl.ds(start, 128)], bufs[b % 2], sems[b % 2]
            )

        pend = fire(0)
        for b in range(BW):
            nxt = fire(b + 1) if b + 1 < BW else None
            pend.wait()
            pend = nxt
            buf = bufs[b % 2]
            q = lax.bitwise_and(docvs[b // L][b % L], 127)
            qv = jnp.full((L,), q, dtype=jnp.int32)
            for kk in range(D // L):
                v = plsc.load_gather(buf, [iota + kk * L, qv])
                flat = b * D + kk * L
                prow[flat // 1024, pl.ds(flat % 1024, L)] = v

        pltpu.sync_copy(prow, out_hbm.at[wid])

    return k(ids, pt)


def _sc_main(ids, pr, w2, ot2, C, S):
    BW = 128
    D = 64
    KI = ids.shape[1]
    CCH = 3
    NCC = pl.cdiv(C, CCH)

    @functools.partial(
        pl.kernel,
        out_type=jax.ShapeDtypeStruct((NW, 8, 128), jnp.float32),
        mesh=_MESH,
        scratch_types=[
            pltpu.VMEM((KI, 128), jnp.int32),
            pltpu.VMEM((C, 128), jnp.int32),
            pltpu.VMEM((S, 128), jnp.int32),
            pltpu.VMEM((BW * D,), jnp.float32),
            pltpu.VMEM((CCH * 128, 128), jnp.float32),
            pltpu.VMEM((CCH * 128, 128), jnp.float32),
            pltpu.VMEM((8, 128), jnp.float32),
            pltpu.SemaphoreType.DMA,
            pltpu.SemaphoreType.DMA,
        ],
        compiler_params=_SC_PARAMS,
    )
    def k(ids_hbm, pr_hbm, w2_hbm, ot2_hbm, out_hbm,
          idsv, cidx, sidx, prow, rowsA, rowsB, res, semA, semB):
        wid = lax.axis_index("c") * NS + lax.axis_index("s")
        iota = lax.iota(jnp.int32, L)

        pltpu.sync_copy(ids_hbm.at[wid], idsv)

        @pl.loop(0, C)
        def _(c):
            @pl.loop(0, 128 // L)
            def _(j):
                w = idsv[1 + c, pl.ds(j * L, L)]
                cidx[c, pl.ds(j * L, L)] = _split_row(w)

        @pl.loop(0, S)
        def _(sj):
            @pl.loop(0, 128 // L)
            def _(j):
                w = idsv[1 + C + sj, pl.ds(j * L, L)]
                sidx[sj, pl.ds(j * L, L)] = _split_row(w)

        bufs = [rowsA, rowsB]
        sems = [semA, semB]

        def fire(idx_ref, base_row, n_rows, parity):
            buf = bufs[parity]
            sem = sems[parity]
            src = w2_hbm if idx_ref is cidx else ot2_hbm
            return [
                pltpu.async_copy(
                    src.at[idx_ref.at[base_row + j]],
                    buf.at[pl.ds(j * 128, 128)],
                    sem,
                )
                for j in range(n_rows)
            ]

        pending = fire(cidx, 0, CCH, 0)

        for r in range(8):
            pltpu.sync_copy(pr_hbm.at[wid, r], prow.at[pl.ds(r * 1024, 1024)])

        for cc in range(NCC):
            nr = min(CCH, C - cc * CCH)
            if cc + 1 < NCC:
                nxt = fire(cidx, (cc + 1) * CCH, min(CCH, C - (cc + 1) * CCH),
                           (cc + 1) % 2)
            else:
                nxt = fire(sidx, 0, min(CCH, S), (cc + 1) % 2)
            for cp in pending:
                cp.wait()
            pending = nxt
            buf = bufs[cc % 2]

            @pl.loop(0, BW // L)
            def _(bg, cc=cc, nr=nr, buf=buf):
                b0 = bg * L
                cvecs = []
                for cl in range(nr):
                    w = idsv[1 + cc * CCH + cl, pl.ds(b0, L)]
                    cvecs.append(_split_col(w))
                for t in range(L):
                    base = (b0 + t) * D
                    acc = [prow[pl.ds(base + kk * L, L)]
                           for kk in range(D // L)]
                    for cl in range(nr):
                        rloc = b0 + t + cl * 128
                        cb = cvecs[cl][t]
                        for kk in range(D // L):
                            acc[kk] = acc[kk] + buf[
                                rloc, pl.ds(cb + kk * L, L)]
                    for kk in range(D // L):
                        prow[pl.ds(base + kk * L, L)] = acc[kk]

        NSC = pl.cdiv(S, CCH)
        g_done = 0
        for sc in range(NSC):
            nr = min(CCH, S - sc * CCH)
            if sc + 1 < NSC:
                nxt = fire(sidx, (sc + 1) * CCH, min(CCH, S - (sc + 1) * CCH),
                           (sc + NCC + 1) % 2)
            else:
                nxt = None
            for cp in pending:
                cp.wait()
            pending = nxt
            buf = bufs[(sc + NCC) % 2]
            ng = nr * (128 // L)

            @pl.loop(0, ng)
            def _(gl, sc=sc, buf=buf, g_done=g_done):
                g = g_done + gl
                r8 = g // (128 // L)
                c8 = g - r8 * (128 // L)
                w = idsv[1 + C + r8, pl.ds(c8 * L, L)]
                colb = _split_col(w)
                ivec = iota + gl * L
                b64 = ((iota + g * L) // S) * D
                r = jnp.zeros((L,), jnp.float32)
                for d in range(D):
                    a = plsc.load_gather(prow, [b64 + d])
                    o = plsc.load_gather(buf, [ivec, colb + d])
                    r = r + a * o
                res[r8, pl.ds(c8 * L, L)] = r

            g_done += ng

        pltpu.sync_copy(res, out_hbm.at[wid])

    return k(ids, pr, w2, ot2)


def kernel(doc_ids, context_ids, sample_ids, paragraph_matrix, word_matrix,
           outputs):
    B, = doc_ids.shape
    C = context_ids.shape[1]
    S = sample_ids.shape[1]
    BW = B // NW
    doc_b = doc_ids.reshape(NW, 1, BW)
    ctx_b = context_ids.reshape(NW, BW, C).transpose(0, 2, 1)
    smp_b = sample_ids.reshape(NW, S, BW)
    pad = jnp.zeros((NW, 32 - 1 - C - S, BW), jnp.int32)
    ids = jnp.concatenate([doc_b, ctx_b, smp_b, pad], axis=1)
    pr = _sc_fetch_p(ids, paragraph_matrix.T)
    w2 = _transpose_pack(word_matrix.T)
    ot2 = _transpose_pack(outputs)
    res = _sc_main(ids, pr, w2, ot2, C, S)
    return res[:, : S * BW // 128, :].reshape(B, S)


# speedup vs baseline: 3.7958x; 1.0878x over previous
import functools

import jax
import jax.numpy as jnp
from jax import lax
from jax.experimental import pallas as pl
from jax.experimental.pallas import tpu as pltpu
from jax.experimental.pallas import tpu_sc as plsc

NC = 2
NS = 16
NW = NC * NS
L = 16
KB = 12


def _transpose_pack(o):
    d, n = o.shape
    hb = 1 << KB
    grid = pl.cdiv(n, 2 * hb)

    def body(x_ref, o_ref):
        t = x_ref[...].T
        o_ref[...] = jnp.concatenate([t[:hb], t[hb:]], axis=1)

    return pl.pallas_call(
        body,
        grid=(grid,),
        in_specs=[pl.BlockSpec((d, 2 * hb), lambda i: (0, i))],
        out_specs=pl.BlockSpec((hb, 2 * d), lambda i: (i, 0)),
        out_shape=jax.ShapeDtypeStruct((grid * hb, 2 * d), o.dtype),
    )(o)


def _split_row(w):
    return (lax.shift_left(lax.shift_right_logical(w, KB + 1), KB)
            + lax.bitwise_and(w, (1 << KB) - 1))


def _split_col(w):
    return lax.shift_left(
        lax.bitwise_and(lax.shift_right_logical(w, KB), 1), 6)


_SC_PARAMS = pltpu.CompilerParams(
    needs_layout_passes=False, use_tc_tiling_on_sc=True
)
_MESH = plsc.VectorSubcoreMesh(core_axis_name="c", subcore_axis_name="s")


def _sc_fetch_p(ids, pt):
    D = pt.shape[0]
    BW = 128

    @functools.partial(
        pl.kernel,
        out_type=jax.ShapeDtypeStruct((NW, 8, 1024), jnp.float32),
        mesh=_MESH,
        scratch_types=[
            pltpu.VMEM((BW,), jnp.int32),
            pltpu.VMEM((D, 128), jnp.float32),
            pltpu.VMEM((D, 128), jnp.float32),
            pltpu.VMEM((D, 128), jnp.float32),
            pltpu.VMEM((D, 128), jnp.float32),
            pltpu.VMEM((D, 128), jnp.float32),
            pltpu.VMEM((D, 128), jnp.float32),
            pltpu.VMEM((D, 128), jnp.float32),
            pltpu.VMEM((D, 128), jnp.float32),
            pltpu.VMEM((8, 1024), jnp.float32),
            pltpu.SemaphoreType.DMA,
            pltpu.SemaphoreType.DMA,
            pltpu.SemaphoreType.DMA,
            pltpu.SemaphoreType.DMA,
            pltpu.SemaphoreType.DMA,
            pltpu.SemaphoreType.DMA,
            pltpu.SemaphoreType.DMA,
            pltpu.SemaphoreType.DMA,
        ],
        compiler_params=_SC_PARAMS,
    )
    def k(ids_hbm, pt_hbm, out_hbm, idv, b0_, b1_, b2_, b3_, b4_, b5_, b6_,
          b7_, prow, s0_, s1_, s2_, s3_, s4_, s5_, s6_, s7_):
        wid = lax.axis_index("c") * NS + lax.axis_index("s")
        iota = lax.iota(jnp.int32, L)
        pltpu.sync_copy(ids_hbm.at[wid, 0], idv)

        NBUF = 8
        bufs = [b0_, b1_, b2_, b3_, b4_, b5_, b6_, b7_]
        sems = [s0_, s1_, s2_, s3_, s4_, s5_, s6_, s7_]
        docvs = [idv[pl.ds(g * L, L)] for g in range(BW // L)]

        def fire(b):
            doc = docvs[b // L][b % L]
            start = pl.multiple_of(lax.bitwise_and(doc, -128), 128)
            return pltpu.async_copy(
                pt_hbm.at[:, pl.ds(start, 128)],
                bufs[b % NBUF], sems[b % NBUF]
            )

        pend = [fire(b) for b in range(NBUF - 1)]
        for b in range(BW):
            if b + NBUF - 1 < BW:
                pend.append(fire(b + NBUF - 1))
            pend.pop(0).wait()
            buf = bufs[b % NBUF]
            q = lax.bitwise_and(docvs[b // L][b % L], 127)
            qv = jnp.full((L,), q, dtype=jnp.int32)
            for kk in range(D // L):
                v = plsc.load_gather(buf, [iota + kk * L, qv])
                flat = b * D + kk * L
                prow[flat // 1024, pl.ds(flat % 1024, L)] = v

        pltpu.sync_copy(prow, out_hbm.at[wid])

    return k(ids, pt)


def _sc_main(ids, pr, w2, ot2, C, S):
    BW = 128
    D = 64
    KI = ids.shape[1]
    CCH = 3
    NCC = pl.cdiv(C, CCH)

    @functools.partial(
        pl.kernel,
        out_type=jax.ShapeDtypeStruct((NW, 8, 128), jnp.float32),
        mesh=_MESH,
        scratch_types=[
            pltpu.VMEM((KI, 128), jnp.int32),
            pltpu.VMEM((C, 128), jnp.int32),
            pltpu.VMEM((S, 128), jnp.int32),
            pltpu.VMEM((BW * D,), jnp.float32),
            pltpu.VMEM((CCH * 128, 128), jnp.float32),
            pltpu.VMEM((CCH * 128, 128), jnp.float32),
            pltpu.VMEM((8, 128), jnp.float32),
            pltpu.SemaphoreType.DMA,
            pltpu.SemaphoreType.DMA,
        ],
        compiler_params=_SC_PARAMS,
    )
    def k(ids_hbm, pr_hbm, w2_hbm, ot2_hbm, out_hbm,
          idsv, cidx, sidx, prow, rowsA, rowsB, res, semA, semB):
        wid = lax.axis_index("c") * NS + lax.axis_index("s")
        iota = lax.iota(jnp.int32, L)

        pltpu.sync_copy(ids_hbm.at[wid], idsv)

        @pl.loop(0, C)
        def _(c):
            @pl.loop(0, 128 // L)
            def _(j):
                w = idsv[1 + c, pl.ds(j * L, L)]
                cidx[c, pl.ds(j * L, L)] = _split_row(w)

        @pl.loop(0, S)
        def _(sj):
            @pl.loop(0, 128 // L)
            def _(j):
                w = idsv[1 + C + sj, pl.ds(j * L, L)]
                sidx[sj, pl.ds(j * L, L)] = _split_row(w)

        bufs = [rowsA, rowsB]
        sems = [semA, semB]

        def fire(idx_ref, base_row, n_rows, parity):
            buf = bufs[parity]
            sem = sems[parity]
            src = w2_hbm if idx_ref is cidx else ot2_hbm
            return [
                pltpu.async_copy(
                    src.at[idx_ref.at[base_row + j]],
                    buf.at[pl.ds(j * 128, 128)],
                    sem,
                )
                for j in range(n_rows)
            ]

        pending = fire(cidx, 0, CCH, 0)

        for r in range(8):
            pltpu.sync_copy(pr_hbm.at[wid, r], prow.at[pl.ds(r * 1024, 1024)])

        for cc in range(NCC):
            nr = min(CCH, C - cc * CCH)
            if cc + 1 < NCC:
                nxt = fire(cidx, (cc + 1) * CCH, min(CCH, C - (cc + 1) * CCH),
                           (cc + 1) % 2)
            else:
                nxt = fire(sidx, 0, min(CCH, S), (cc + 1) % 2)
            for cp in pending:
                cp.wait()
            pending = nxt
            buf = bufs[cc % 2]

            @pl.loop(0, BW // L)
            def _(bg, cc=cc, nr=nr, buf=buf):
                b0 = bg * L
                cvecs = []
                for cl in range(nr):
                    w = idsv[1 + cc * CCH + cl, pl.ds(b0, L)]
                    cvecs.append(_split_col(w))
                for t in range(L):
                    base = (b0 + t) * D
                    acc = [prow[pl.ds(base + kk * L, L)]
                           for kk in range(D // L)]
                    for cl in range(nr):
                        rloc = b0 + t + cl * 128
                        cb = cvecs[cl][t]
                        for kk in range(D // L):
                            acc[kk] = acc[kk] + buf[
                                rloc, pl.ds(cb + kk * L, L)]
                    for kk in range(D // L):
                        prow[pl.ds(base + kk * L, L)] = acc[kk]

        NSC = pl.cdiv(S, CCH)
        g_done = 0
        for sc in range(NSC):
            nr = min(CCH, S - sc * CCH)
            if sc + 1 < NSC:
                nxt = fire(sidx, (sc + 1) * CCH, min(CCH, S - (sc + 1) * CCH),
                           (sc + NCC + 1) % 2)
            else:
                nxt = None
            for cp in pending:
                cp.wait()
            pending = nxt
            buf = bufs[(sc + NCC) % 2]
            ng = nr * (128 // L)

            @pl.loop(0, ng)
            def _(gl, sc=sc, buf=buf, g_done=g_done):
                g = g_done + gl
                r8 = g // (128 // L)
                c8 = g - r8 * (128 // L)
                w = idsv[1 + C + r8, pl.ds(c8 * L, L)]
                colb = _split_col(w)
                ivec = iota + gl * L
                b64 = ((iota + g * L) // S) * D
                r = jnp.zeros((L,), jnp.float32)
                for d in range(D):
                    a = plsc.load_gather(prow, [b64 + d])
                    o = plsc.load_gather(buf, [ivec, colb + d])
                    r = r + a * o
                res[r8, pl.ds(c8 * L, L)] = r

            g_done += ng

        pltpu.sync_copy(res, out_hbm.at[wid])

    return k(ids, pr, w2, ot2)


def kernel(doc_ids, context_ids, sample_ids, paragraph_matrix, word_matrix,
           outputs):
    B, = doc_ids.shape
    C = context_ids.shape[1]
    S = sample_ids.shape[1]
    BW = B // NW
    doc_b = doc_ids.reshape(NW, 1, BW)
    ctx_b = context_ids.reshape(NW, BW, C).transpose(0, 2, 1)
    smp_b = sample_ids.reshape(NW, S, BW)
    pad = jnp.zeros((NW, 32 - 1 - C - S, BW), jnp.int32)
    ids = jnp.concatenate([doc_b, ctx_b, smp_b, pad], axis=1)
    pr = _sc_fetch_p(ids, paragraph_matrix.T)
    w2 = _transpose_pack(word_matrix.T)
    ot2 = _transpose_pack(outputs)
    res = _sc_main(ids, pr, w2, ot2, C, S)
    return res[:, : S * BW // 128, :].reshape(B, S)


# speedup vs baseline: 3.8888x; 1.0245x over previous
import functools

import jax
import jax.numpy as jnp
from jax import lax
from jax.experimental import pallas as pl
from jax.experimental.pallas import tpu as pltpu
from jax.experimental.pallas import tpu_sc as plsc

NC = 2
NS = 16
NW = NC * NS
L = 16
KB = 12


def _transpose_pack(o):
    d, n = o.shape
    hb = 1 << KB
    grid = pl.cdiv(n, 2 * hb)

    def body(x_ref, o_ref):
        t = x_ref[...].T
        o_ref[...] = jnp.concatenate([t[:hb], t[hb:]], axis=1)

    return pl.pallas_call(
        body,
        grid=(grid,),
        in_specs=[pl.BlockSpec((d, 2 * hb), lambda i: (0, i))],
        out_specs=pl.BlockSpec((hb, 2 * d), lambda i: (i, 0)),
        out_shape=jax.ShapeDtypeStruct((grid * hb, 2 * d), o.dtype),
    )(o)


def _split_row(w):
    return (lax.shift_left(lax.shift_right_logical(w, KB + 1), KB)
            + lax.bitwise_and(w, (1 << KB) - 1))


def _split_col(w):
    return lax.shift_left(
        lax.bitwise_and(lax.shift_right_logical(w, KB), 1), 6)


_SC_PARAMS = pltpu.CompilerParams(
    needs_layout_passes=False, use_tc_tiling_on_sc=True
)
_MESH = plsc.VectorSubcoreMesh(core_axis_name="c", subcore_axis_name="s")


def _sc_fetch_p(ids, pt):
    D = pt.shape[0]
    BW = 128

    @functools.partial(
        pl.kernel,
        out_type=jax.ShapeDtypeStruct((NW, 8, 1024), jnp.float32),
        mesh=_MESH,
        scratch_types=[
            pltpu.VMEM((BW,), jnp.int32),
            pltpu.VMEM((D, 128), jnp.float32),
            pltpu.VMEM((D, 128), jnp.float32),
            pltpu.VMEM((D, 128), jnp.float32),
            pltpu.VMEM((D, 128), jnp.float32),
            pltpu.VMEM((D, 128), jnp.float32),
            pltpu.VMEM((D, 128), jnp.float32),
            pltpu.VMEM((D, 128), jnp.float32),
            pltpu.VMEM((D, 128), jnp.float32),
            pltpu.VMEM((D, 128), jnp.float32),
            pltpu.VMEM((D, 128), jnp.float32),
            pltpu.VMEM((D, 128), jnp.float32),
            pltpu.VMEM((D, 128), jnp.float32),
            pltpu.VMEM((8, 1024), jnp.float32),
            pltpu.SemaphoreType.DMA,
            pltpu.SemaphoreType.DMA,
            pltpu.SemaphoreType.DMA,
            pltpu.SemaphoreType.DMA,
            pltpu.SemaphoreType.DMA,
            pltpu.SemaphoreType.DMA,
            pltpu.SemaphoreType.DMA,
            pltpu.SemaphoreType.DMA,
            pltpu.SemaphoreType.DMA,
            pltpu.SemaphoreType.DMA,
            pltpu.SemaphoreType.DMA,
            pltpu.SemaphoreType.DMA,
        ],
        compiler_params=_SC_PARAMS,
    )
    def k(ids_hbm, pt_hbm, out_hbm, idv, b0_, b1_, b2_, b3_, b4_, b5_, b6_,
          b7_, b8_, b9_, b10_, b11_, prow, s0_, s1_, s2_, s3_, s4_, s5_,
          s6_, s7_, s8_, s9_, s10_, s11_):
        wid = lax.axis_index("c") * NS + lax.axis_index("s")
        iota = lax.iota(jnp.int32, L)
        pltpu.sync_copy(ids_hbm.at[wid, 0], idv)

        NBUF = 12
        bufs = [b0_, b1_, b2_, b3_, b4_, b5_, b6_, b7_, b8_, b9_, b10_, b11_]
        sems = [s0_, s1_, s2_, s3_, s4_, s5_, s6_, s7_, s8_, s9_, s10_, s11_]
        docvs = [idv[pl.ds(g * L, L)] for g in range(BW // L)]

        def fire(b):
            doc = docvs[b // L][b % L]
            start = pl.multiple_of(lax.bitwise_and(doc, -128), 128)
            return pltpu.async_copy(
                pt_hbm.at[:, pl.ds(start, 128)],
                bufs[b % NBUF], sems[b % NBUF]
            )

        pend = [fire(b) for b in range(NBUF - 1)]
        for b in range(BW):
            if b + NBUF - 1 < BW:
                pend.append(fire(b + NBUF - 1))
            pend.pop(0).wait()
            buf = bufs[b % NBUF]
            q = lax.bitwise_and(docvs[b // L][b % L], 127)
            qv = jnp.full((L,), q, dtype=jnp.int32)
            for kk in range(D // L):
                v = plsc.load_gather(buf, [iota + kk * L, qv])
                flat = b * D + kk * L
                prow[flat // 1024, pl.ds(flat % 1024, L)] = v

        pltpu.sync_copy(prow, out_hbm.at[wid])

    return k(ids, pt)


def _sc_main(ids, pr, w2, ot2, C, S):
    BW = 128
    D = 64
    KI = ids.shape[1]
    CCH = 3
    NCC = pl.cdiv(C, CCH)

    @functools.partial(
        pl.kernel,
        out_type=jax.ShapeDtypeStruct((NW, 8, 128), jnp.float32),
        mesh=_MESH,
        scratch_types=[
            pltpu.VMEM((KI, 128), jnp.int32),
            pltpu.VMEM((C, 128), jnp.int32),
            pltpu.VMEM((S, 128), jnp.int32),
            pltpu.VMEM((BW * D,), jnp.float32),
            pltpu.VMEM((CCH * 128, 128), jnp.float32),
            pltpu.VMEM((CCH * 128, 128), jnp.float32),
            pltpu.VMEM((8, 128), jnp.float32),
            pltpu.SemaphoreType.DMA,
            pltpu.SemaphoreType.DMA,
        ],
        compiler_params=_SC_PARAMS,
    )
    def k(ids_hbm, pr_hbm, w2_hbm, ot2_hbm, out_hbm,
          idsv, cidx, sidx, prow, rowsA, rowsB, res, semA, semB):
        wid = lax.axis_index("c") * NS + lax.axis_index("s")
        iota = lax.iota(jnp.int32, L)

        pltpu.sync_copy(ids_hbm.at[wid], idsv)

        @pl.loop(0, C)
        def _(c):
            @pl.loop(0, 128 // L)
            def _(j):
                w = idsv[1 + c, pl.ds(j * L, L)]
                cidx[c, pl.ds(j * L, L)] = _split_row(w)

        @pl.loop(0, S)
        def _(sj):
            @pl.loop(0, 128 // L)
            def _(j):
                w = idsv[1 + C + sj, pl.ds(j * L, L)]
                sidx[sj, pl.ds(j * L, L)] = _split_row(w)

        bufs = [rowsA, rowsB]
        sems = [semA, semB]

        def fire(idx_ref, base_row, n_rows, parity):
            buf = bufs[parity]
            sem = sems[parity]
            src = w2_hbm if idx_ref is cidx else ot2_hbm
            return [
                pltpu.async_copy(
                    src.at[idx_ref.at[base_row + j]],
                    buf.at[pl.ds(j * 128, 128)],
                    sem,
                )
                for j in range(n_rows)
            ]

        pending = fire(cidx, 0, CCH, 0)

        prcps = [
            pltpu.async_copy(pr_hbm.at[wid, r],
                             prow.at[pl.ds(r * 1024, 1024)], semA)
            for r in range(8)
        ]
        for cp in prcps:
            cp.wait()

        for cc in range(NCC):
            nr = min(CCH, C - cc * CCH)
            if cc + 1 < NCC:
                nxt = fire(cidx, (cc + 1) * CCH, min(CCH, C - (cc + 1) * CCH),
                           (cc + 1) % 2)
            else:
                nxt = fire(sidx, 0, min(CCH, S), (cc + 1) % 2)
            for cp in pending:
                cp.wait()
            pending = nxt
            buf = bufs[cc % 2]

            @pl.loop(0, BW // L)
            def _(bg, cc=cc, nr=nr, buf=buf):
                b0 = bg * L
                cvecs = []
                for cl in range(nr):
                    w = idsv[1 + cc * CCH + cl, pl.ds(b0, L)]
                    cvecs.append(_split_col(w))
                for t in range(L):
                    base = (b0 + t) * D
                    acc = [prow[pl.ds(base + kk * L, L)]
                           for kk in range(D // L)]
                    for cl in range(nr):
                        rloc = b0 + t + cl * 128
                        cb = cvecs[cl][t]
                        for kk in range(D // L):
                            acc[kk] = acc[kk] + buf[
                                rloc, pl.ds(cb + kk * L, L)]
                    for kk in range(D // L):
                        prow[pl.ds(base + kk * L, L)] = acc[kk]

        NSC = pl.cdiv(S, CCH)
        g_done = 0
        for sc in range(NSC):
            nr = min(CCH, S - sc * CCH)
            if sc + 1 < NSC:
                nxt = fire(sidx, (sc + 1) * CCH, min(CCH, S - (sc + 1) * CCH),
                           (sc + NCC + 1) % 2)
            else:
                nxt = None
            for cp in pending:
                cp.wait()
            pending = nxt
            buf = bufs[(sc + NCC) % 2]
            ng = nr * (128 // L)

            @pl.loop(0, ng)
            def _(gl, sc=sc, buf=buf, g_done=g_done):
                g = g_done + gl
                r8 = g // (128 // L)
                c8 = g - r8 * (128 // L)
                w = idsv[1 + C + r8, pl.ds(c8 * L, L)]
                colb = _split_col(w)
                ivec = iota + gl * L
                b64 = ((iota + g * L) // S) * D
                r = jnp.zeros((L,), jnp.float32)
                for d in range(D):
                    a = plsc.load_gather(prow, [b64 + d])
                    o = plsc.load_gather(buf, [ivec, colb + d])
                    r = r + a * o
                res[r8, pl.ds(c8 * L, L)] = r

            g_done += ng

        pltpu.sync_copy(res, out_hbm.at[wid])

    return k(ids, pr, w2, ot2)


def kernel(doc_ids, context_ids, sample_ids, paragraph_matrix, word_matrix,
           outputs):
    B, = doc_ids.shape
    C = context_ids.shape[1]
    S = sample_ids.shape[1]
    BW = B // NW
    doc_b = doc_ids.reshape(NW, 1, BW)
    ctx_b = context_ids.reshape(NW, BW, C).transpose(0, 2, 1)
    smp_b = sample_ids.reshape(NW, S, BW)
    pad = jnp.zeros((NW, 32 - 1 - C - S, BW), jnp.int32)
    ids = jnp.concatenate([doc_b, ctx_b, smp_b, pad], axis=1)
    pr = _sc_fetch_p(ids, paragraph_matrix.T)
    w2 = _transpose_pack(word_matrix.T)
    ot2 = _transpose_pack(outputs)
    res = _sc_main(ids, pr, w2, ot2, C, S)
    return res[:, : S * BW // 128, :].reshape(B, S)
